# baseline (speedup 1.0000x reference)
import jax
import jax.numpy as jnp
from jax.experimental import pallas as pl
from jax.experimental.pallas import tpu as pltpu


_BM = 80
_NS = 5


def _mm_kernel(f_ref, *refs):
    a_refs = refs[:_NS]
    o_ref = refs[_NS]
    f16_scr = refs[_NS + 1]

    @pl.when(pl.program_id(0) == 0)
    def _():
        f16_scr[...] = f_ref[...].astype(jnp.bfloat16)

    f16 = f16_scr[...]
    for s, a_ref in enumerate(a_refs):
        o_ref[s * _BM:(s + 1) * _BM, :] = jax.lax.dot_general(
            a_ref[...], f16,
            (((1,), (0,)), ((), ())),
            precision=jax.lax.Precision.DEFAULT,
            preferred_element_type=jnp.float32,
        )


def kernel(features, A):
    m, k = A.shape
    d = features.shape[1]
    a_specs = [
        pl.BlockSpec((_BM, k), lambda i, s=s: (_NS * i + s, 0))
        for s in range(_NS)
    ]
    return pl.pallas_call(
        _mm_kernel,
        grid=(m // (_NS * _BM),),
        in_specs=[pl.BlockSpec((k, d), lambda i: (0, 0))] + a_specs,
        out_specs=pl.BlockSpec((_NS * _BM, d), lambda i: (i, 0)),
        out_shape=jax.ShapeDtypeStruct((m, d), jnp.float32),
        scratch_shapes=[pltpu.VMEM((k, d), jnp.bfloat16)],
    )(features, *([A] * _NS))


# speedup vs baseline: 1.0324x; 1.0324x over previous
import jax
import jax.numpy as jnp
from jax.experimental import pallas as pl
from jax.experimental.pallas import tpu as pltpu


_BM = 400


def _mm_kernel(f_ref, a_ref, o_ref, f16_scr):
    @pl.when(pl.program_id(0) == 0)
    def _():
        f16_scr[...] = f_ref[...].astype(jnp.bfloat16)

    o_ref[...] = jax.lax.dot_general(
        a_ref[...], f16_scr[...],
        (((1,), (0,)), ((), ())),
        precision=jax.lax.Precision.DEFAULT,
        preferred_element_type=jnp.float32,
    )


def kernel(features, A):
    m, k = A.shape
    d = features.shape[1]
    return pl.pallas_call(
        _mm_kernel,
        grid=(m // _BM,),
        in_specs=[
            pl.BlockSpec((k, d), lambda i: (0, 0)),
            pl.BlockSpec((_BM, k), lambda i: (i, 0)),
        ],
        out_specs=pl.BlockSpec((_BM, d), lambda i: (i, 0)),
        out_shape=jax.ShapeDtypeStruct((m, d), jnp.float32),
        scratch_shapes=[pltpu.VMEM((k, d), jnp.bfloat16)],
    )(features, A)
